# Initial kernel scaffold; baseline (speedup 1.0000x reference)
import jax
import jax.numpy as jnp
from jax.experimental import pallas as pl


def kernel(x, edge_index_r0, edge_index_r1, conv_Wl, conv_Wr, conv_b, outp_W, outp_b, attn_Wq, attn_Wk, attn_Wv, attn_Wo, attn_bq, attn_bk, attn_bv, attn_bo, mlp_W1, mlp_b1, mlp_W2, mlp_b2):
    raise NotImplementedError("write your pallas kernel here")


# speedup vs baseline: 3.4539x; 3.4539x over previous
import functools
import jax
import jax.numpy as jnp
from jax import lax
from jax.experimental import pallas as pl
from jax.experimental.pallas import tpu as pltpu
from jax.experimental.pallas import tpu_sc as plsc

_N = 50000
_H = 64
_E = 800000
_NC = 2
_NS = 16
_NSEG = 25088
_NP = _NC * _NSEG
_TRASH = _NSEG
_ACC_ROWS = _NSEG + 8
_TILE_ROWS = _NSEG // _NS
_ZROWS = 49
_EROWS = 6272
_EPAD = _EROWS * 128
_RPT = _EROWS // _NS
_NJ = 2
_OUTER = _RPT // _NJ


def _make_segsum():
    mesh = plsc.VectorSubcoreMesh(
        core_axis_name="c", subcore_axis_name="s",
        num_cores=_NC, num_subcores=_NS)
    out_type = (
        jax.ShapeDtypeStruct((_NP, _H), jnp.float32),
        jax.ShapeDtypeStruct((_NP,), jnp.float32),
    )
    scratch = [
        pltpu.VMEM((_NJ, 128), jnp.int32),
        pltpu.VMEM((_NJ, 128), jnp.int32),
        pltpu.VMEM((_NJ, 128), jnp.int32),
        pltpu.VMEM((_NJ, 128, _H), jnp.float32),
        pltpu.VMEM((128,), jnp.float32),
        pltpu.VMEM((_ZROWS, _H), jnp.float32),
        pltpu.VMEM((_TILE_ROWS,), jnp.float32),
        pltpu.VMEM_SHARED((_ACC_ROWS, _H), jnp.float32),
        pltpu.VMEM_SHARED((_ACC_ROWS,), jnp.float32),
        pltpu.SemaphoreType.DMA,
    ]

    @functools.partial(pl.kernel, out_type=out_type, mesh=mesh,
                       scratch_types=scratch,
                       compiler_params=pltpu.CompilerParams(
                           use_tc_tiling_on_sc=False))
    def segsum(h, src, dst, zrow, zcnt, onesc, sums, cnt,
               srcbuf, dstbuf, locbuf, rows, ones, zbuf, zcbuf,
               acc, cntacc, sem):
        cid = lax.axis_index("c")
        sid = lax.axis_index("s")
        base = cid * _NSEG

        pltpu.sync_copy(onesc, ones)
        pltpu.sync_copy(zrow, zbuf)
        pltpu.sync_copy(zcnt, zcbuf)
        for k0 in range(_TILE_ROWS // _ZROWS):
            pltpu.sync_copy(
                zbuf, acc.at[pl.ds(sid * _TILE_ROWS + k0 * _ZROWS, _ZROWS)])
        pltpu.sync_copy(zcbuf, cntacc.at[pl.ds(sid * _TILE_ROWS, _TILE_ROWS)])

        @pl.when(sid == 0)
        def _():
            pltpu.sync_copy(zbuf.at[pl.ds(0, 8)], acc.at[pl.ds(_NSEG, 8)])
            pltpu.sync_copy(zcbuf.at[pl.ds(0, 8)], cntacc.at[pl.ds(_NSEG, 8)])

        plsc.subcore_barrier()

        def body(i, carry):
            row0 = sid * _RPT + i * _NJ
            pltpu.sync_copy(src.at[pl.ds(row0, _NJ)], srcbuf)
            pltpu.sync_copy(dst.at[pl.ds(row0, _NJ)], dstbuf)
            for r in range(_NJ):
                for g in range(128 // 16):
                    d = dstbuf[r, pl.ds(g * 16, 16)]
                    loc = d - base
                    owned = (loc >= 0) & (loc < _NSEG)
                    locbuf[r, pl.ds(g * 16, 16)] = jnp.where(owned, loc, _TRASH)
            cps = [pltpu.async_copy(h.at[srcbuf.at[r]], rows.at[r], sem)
                   for r in range(_NJ)]
            for cp in cps:
                cp.wait()
            for r in range(_NJ):
                pltpu.sync_copy(rows.at[r], acc.at[locbuf.at[r]], add=True)
                pltpu.sync_copy(ones, cntacc.at[locbuf.at[r]], add=True)
            return carry

        lax.fori_loop(0, _OUTER, body, 0)

        plsc.subcore_barrier()
        off = sid * _TILE_ROWS
        pltpu.sync_copy(acc.at[pl.ds(off, _TILE_ROWS)],
                        sums.at[pl.ds(base + off, _TILE_ROWS)])
        pltpu.sync_copy(cntacc.at[pl.ds(off, _TILE_ROWS)],
                        cnt.at[pl.ds(base + off, _TILE_ROWS)])

    return segsum


_BLK = 3584


def _row_spec():
    return pl.BlockSpec((_BLK, _H), lambda i: (i, 0))


def _col_spec():
    return pl.BlockSpec((_BLK, 1), lambda i: (i, 0))


def _w_spec():
    return pl.BlockSpec((_H, _H), lambda i: (0, 0))


def _b_spec():
    return pl.BlockSpec((1, _H), lambda i: (0, 0))


def _dot(a, b):
    return jnp.dot(a, b, preferred_element_type=jnp.float32)


def _layer0_body(x_r, s0_r, c0_r, s1_r, c1_r,
                 wl0, wr0, bb0, wl1, wr1, bb1, h0_r, h1_r):
    x = x_r[...]
    agg0 = s0_r[...] * (1.0 / jnp.maximum(c0_r[...], 1.0))
    agg1 = s1_r[...] * (1.0 / jnp.maximum(c1_r[...], 1.0))
    h0_r[...] = jnp.maximum(
        _dot(agg0, wl0[...]) + bb0[...] + _dot(x, wr0[...]), 0.0)
    h1_r[...] = jnp.maximum(
        _dot(agg1, wl1[...]) + bb1[...] + _dot(x, wr1[...]), 0.0)


def _tc_layer0(xp, s00, c0, s10, c1, wl0, wr0, b0, wl1, wr1, b1):
    grid = (_NP // _BLK,)
    return pl.pallas_call(
        _layer0_body,
        grid=grid,
        in_specs=[_row_spec(), _row_spec(), _col_spec(), _row_spec(),
                  _col_spec(), _w_spec(), _w_spec(), _b_spec(),
                  _w_spec(), _w_spec(), _b_spec()],
        out_specs=[_row_spec(), _row_spec()],
        out_shape=[jax.ShapeDtypeStruct((_NP, _H), jnp.float32),
                   jax.ShapeDtypeStruct((_NP, _H), jnp.float32)],
    )(xp, s00, c0, s10, c1, wl0, wr0, b0, wl1, wr1, b1)


def _final_body(h0_r, h1_r, sA_r, cA_r, sB_r, cB_r,
                wl0, wr0, bb0, wl1, wr1, bb1,
                op0, ob0, op1, ob1,
                wq, bq_r, wk, bk_r, wv, bv_r, wo, bo_r,
                w1, b1_r, w2, b2_r, out_r):
    aggA = sA_r[...] * (1.0 / jnp.maximum(cA_r[...], 1.0))
    aggB = sB_r[...] * (1.0 / jnp.maximum(cB_r[...], 1.0))
    g0 = jnp.maximum(
        _dot(aggA, wl0[...]) + bb0[...] + _dot(h0_r[...], wr0[...]), 0.0)
    g1 = jnp.maximum(
        _dot(aggB, wl1[...]) + bb1[...] + _dot(h1_r[...], wr1[...]), 0.0)
    xw0 = (_dot(g0, op0[...]) + ob0[...]) * 0.5
    xw1 = (_dot(g1, op1[...]) + ob1[...]) * 0.5
    q0 = _dot(xw0, wq[...]) + bq_r[...]
    q1 = _dot(xw1, wq[...]) + bq_r[...]
    k0 = _dot(xw0, wk[...]) + bk_r[...]
    k1 = _dot(xw1, wk[...]) + bk_r[...]
    v0 = _dot(xw0, wv[...]) + bv_r[...]
    v1 = _dot(xw1, wv[...]) + bv_r[...]
    rr = lax.broadcasted_iota(jnp.int32, (_H, 4), 0)
    cc = lax.broadcasted_iota(jnp.int32, (_H, 4), 1)
    hsel = (rr // 16 == cc).astype(jnp.float32)
    hexp = (lax.broadcasted_iota(jnp.int32, (4, _H), 0) ==
            lax.broadcasted_iota(jnp.int32, (4, _H), 1) // 16
            ).astype(jnp.float32)
    scale = 0.25
    s00 = _dot(q0 * k0, hsel) * scale
    s01 = _dot(q0 * k1, hsel) * scale
    s10 = _dot(q1 * k0, hsel) * scale
    s11 = _dot(q1 * k1, hsel) * scale
    m0 = jnp.maximum(s00, s01)
    e00 = jnp.exp(s00 - m0)
    e01 = jnp.exp(s01 - m0)
    d0 = e00 + e01
    m1 = jnp.maximum(s10, s11)
    e10 = jnp.exp(s10 - m1)
    e11 = jnp.exp(s11 - m1)
    d1 = e10 + e11
    o0 = _dot(e00 / d0, hexp) * v0 + _dot(e01 / d0, hexp) * v1
    o1 = _dot(e10 / d1, hexp) * v0 + _dot(e11 / d1, hexp) * v1
    ao0 = _dot(o0, wo[...]) + bo_r[...]
    ao1 = _dot(o1, wo[...]) + bo_r[...]
    pooled = (ao0 + ao1) * 0.5
    hmid = jnp.maximum(_dot(pooled, w1[...]) + b1_r[...], 0.0)
    out_r[...] = _dot(hmid, w2[...]) + b2_r[...]


def _tc_final(h0, h1, sA, cA, sB, cB, wl0, wr0, b0, wl1, wr1, b1,
              op0, ob0, op1, ob1, wq, bq, wk, bk, wv, bv, wo, bo,
              w1, b1m, w2, b2):
    grid = (_NP // _BLK,)
    w2spec = pl.BlockSpec((_H, 1), lambda i: (0, 0))
    b2spec = pl.BlockSpec((1, 1), lambda i: (0, 0))
    return pl.pallas_call(
        _final_body,
        grid=grid,
        in_specs=[_row_spec(), _row_spec(), _row_spec(), _col_spec(),
                  _row_spec(), _col_spec(),
                  _w_spec(), _w_spec(), _b_spec(),
                  _w_spec(), _w_spec(), _b_spec(),
                  _w_spec(), _b_spec(), _w_spec(), _b_spec(),
                  _w_spec(), _b_spec(), _w_spec(), _b_spec(),
                  _w_spec(), _b_spec(), _w_spec(), _b_spec(),
                  _w_spec(), _b_spec(), w2spec, b2spec],
        out_specs=[_col_spec()],
        out_shape=[jax.ShapeDtypeStruct((_NP, 1), jnp.float32)],
    )(h0, h1, sA, cA, sB, cB, wl0, wr0, b0, wl1, wr1, b1,
      op0, ob0, op1, ob1, wq, bq, wk, bk, wv, bv, wo, bo,
      w1, b1m, w2, b2)[0]


def _prep_edges(ei):
    src = jnp.concatenate(
        [ei[0], jnp.zeros((_EPAD - _E,), dtype=jnp.int32)]).reshape(_EROWS, 128)
    dst = jnp.concatenate(
        [ei[1], jnp.full((_EPAD - _E,), jnp.int32(1 << 20))]).reshape(
            _EROWS, 128)
    return src, dst


def kernel(x, edge_index_r0, edge_index_r1, conv_Wl, conv_Wr, conv_b,
           outp_W, outp_b, attn_Wq, attn_Wk, attn_Wv, attn_Wo,
           attn_bq, attn_bk, attn_bv, attn_bo,
           mlp_W1, mlp_b1, mlp_W2, mlp_b2):
    segsum = _make_segsum()
    s0, d0 = _prep_edges(edge_index_r0)
    s1, d1 = _prep_edges(edge_index_r1)
    xp = jnp.concatenate(
        [x, jnp.zeros((_NP - _N, _H), dtype=jnp.float32)], axis=0)
    zrow = jnp.zeros((_ZROWS, _H), dtype=jnp.float32)
    zcnt = jnp.zeros((_TILE_ROWS,), dtype=jnp.float32)
    onesc = jnp.ones((128,), dtype=jnp.float32)

    sums00, cnt0 = segsum(xp, s0, d0, zrow, zcnt, onesc)
    sums10, cnt1 = segsum(xp, s1, d1, zrow, zcnt, onesc)
    c0 = cnt0.reshape(_NP, 1)
    c1 = cnt1.reshape(_NP, 1)
    h0, h1 = _tc_layer0(
        xp, sums00, c0, sums10, c1,
        conv_Wl[0, 0].T, conv_Wr[0, 0].T, conv_b[0, 0].reshape(1, _H),
        conv_Wl[1, 0].T, conv_Wr[1, 0].T, conv_b[1, 0].reshape(1, _H))

    sums01, _ = segsum(h0, s1, d1, zrow, zcnt, onesc)
    sums11, _ = segsum(h1, s0, d0, zrow, zcnt, onesc)

    out = _tc_final(
        h0, h1, sums01, c1, sums11, c0,
        conv_Wl[0, 1].T, conv_Wr[0, 1].T, conv_b[0, 1].reshape(1, _H),
        conv_Wl[1, 1].T, conv_Wr[1, 1].T, conv_b[1, 1].reshape(1, _H),
        outp_W[0].T, outp_b[0].reshape(1, _H),
        outp_W[1].T, outp_b[1].reshape(1, _H),
        attn_Wq.T, attn_bq.reshape(1, _H),
        attn_Wk.T, attn_bk.reshape(1, _H),
        attn_Wv.T, attn_bv.reshape(1, _H),
        attn_Wo.T, attn_bo.reshape(1, _H),
        mlp_W1.T, mlp_b1.reshape(1, _H),
        mlp_W2.T, mlp_b2.reshape(1, 1))
    return out[:_N, 0]


# speedup vs baseline: 5.4215x; 1.5697x over previous
import functools
import jax
import jax.numpy as jnp
from jax import lax
from jax.experimental import pallas as pl
from jax.experimental.pallas import tpu as pltpu
from jax.experimental.pallas import tpu_sc as plsc

_N = 50000
_H = 64
_HH = 32
_E = 800000
_NC = 2
_NS = 16
_NP = 50176
_TRASH = _NP
_ACC_ROWS = _NP + 8
_TILE_ROWS = _NP // _NS
_ZROWS = 64
_EROWS = 6272
_EPAD = _EROWS * 128
_RPT = _EROWS // _NS
_NJ = 2
_OUTER = _RPT // _NJ


def _make_segsum(with_counts):
    mesh = plsc.VectorSubcoreMesh(
        core_axis_name="c", subcore_axis_name="s",
        num_cores=_NC, num_subcores=_NS)
    out_type = [jax.ShapeDtypeStruct((_NC * _NP, _HH), jnp.float32)]
    if with_counts:
        out_type.append(jax.ShapeDtypeStruct((_NP,), jnp.float32))
    scratch = [
        pltpu.VMEM((_NJ, 128), jnp.int32),
        pltpu.VMEM((_NJ, 128), jnp.int32),
        pltpu.VMEM((_NJ, 128, _HH), jnp.float32),
        pltpu.VMEM((_ZROWS, _HH), jnp.float32),
        pltpu.VMEM_SHARED((_ACC_ROWS, _HH), jnp.float32),
        pltpu.SemaphoreType.DMA,
    ]
    if with_counts:
        scratch += [
            pltpu.VMEM((128,), jnp.float32),
            pltpu.VMEM((_TILE_ROWS,), jnp.float32),
            pltpu.VMEM_SHARED((_ACC_ROWS,), jnp.float32),
        ]

    @functools.partial(pl.kernel, out_type=tuple(out_type), mesh=mesh,
                       scratch_types=scratch,
                       compiler_params=pltpu.CompilerParams(
                           use_tc_tiling_on_sc=False))
    def segsum(*refs):
        if with_counts:
            (h_lo, h_hi, src, dst, zrow, zcnt, onesc, sums, cnt,
             srcbuf, dstbuf, rows, zbuf, acc, sem, ones, zcbuf, cntacc) = refs
        else:
            (h_lo, h_hi, src, dst, zrow, sums,
             srcbuf, dstbuf, rows, zbuf, acc, sem) = refs
        cid = lax.axis_index("c")
        sid = lax.axis_index("s")

        pltpu.sync_copy(zrow, zbuf)
        for k0 in range(_TILE_ROWS // _ZROWS):
            pltpu.sync_copy(
                zbuf, acc.at[pl.ds(sid * _TILE_ROWS + k0 * _ZROWS, _ZROWS)])
        if with_counts:
            pltpu.sync_copy(onesc, ones)
            pltpu.sync_copy(zcnt, zcbuf)
            pltpu.sync_copy(zcbuf,
                            cntacc.at[pl.ds(sid * _TILE_ROWS, _TILE_ROWS)])

        @pl.when(sid == 0)
        def _():
            pltpu.sync_copy(zbuf.at[pl.ds(0, 8)], acc.at[pl.ds(_NP, 8)])
            if with_counts:
                pltpu.sync_copy(zcbuf.at[pl.ds(0, 8)],
                                cntacc.at[pl.ds(_NP, 8)])

        plsc.subcore_barrier()

        def make_body(h):
            def body(i, carry):
                row0 = sid * _RPT + i * _NJ
                pltpu.sync_copy(src.at[pl.ds(row0, _NJ)], srcbuf)
                pltpu.sync_copy(dst.at[pl.ds(row0, _NJ)], dstbuf)
                cps = [pltpu.async_copy(h.at[srcbuf.at[r]], rows.at[r], sem)
                       for r in range(_NJ)]
                for cp in cps:
                    cp.wait()
                for r in range(_NJ):
                    pltpu.sync_copy(rows.at[r], acc.at[dstbuf.at[r]], add=True)
                    if with_counts:
                        pltpu.sync_copy(ones, cntacc.at[dstbuf.at[r]],
                                        add=True)
                return carry
            return body

        @pl.when(cid == 0)
        def _():
            lax.fori_loop(0, _OUTER, make_body(h_lo), 0)

        @pl.when(cid == 1)
        def _():
            lax.fori_loop(0, _OUTER, make_body(h_hi), 0)

        plsc.subcore_barrier()
        off = sid * _TILE_ROWS
        pltpu.sync_copy(acc.at[pl.ds(off, _TILE_ROWS)],
                        sums.at[pl.ds(cid * _NP + off, _TILE_ROWS)])
        if with_counts:
            @pl.when(cid == 0)
            def _():
                pltpu.sync_copy(cntacc.at[pl.ds(off, _TILE_ROWS)],
                                cnt.at[pl.ds(off, _TILE_ROWS)])

    return segsum


_BLK = 3584
_NBLK = _NP // _BLK


def _row_spec():
    return pl.BlockSpec((_BLK, _H), lambda i: (i, 0))


def _half_spec(half):
    return pl.BlockSpec((_BLK, _HH), lambda i: (i + half * _NBLK, 0))


def _out_half_spec():
    return pl.BlockSpec((_BLK, _HH), lambda i: (i, 0))


def _col_spec():
    return pl.BlockSpec((_BLK, 1), lambda i: (i, 0))


def _w_spec():
    return pl.BlockSpec((_H, _H), lambda i: (0, 0))


def _b_spec():
    return pl.BlockSpec((1, _H), lambda i: (0, 0))


def _dot(a, b):
    return jnp.dot(a, b, preferred_element_type=jnp.float32)


def _layer0_body(x_r, s_lo, s_hi, c_r, wl, wr, bb, h_lo, h_hi):
    agg = jnp.concatenate([s_lo[...], s_hi[...]], axis=1)
    agg = agg * (1.0 / jnp.maximum(c_r[...], 1.0))
    h = jnp.maximum(
        _dot(agg, wl[...]) + bb[...] + _dot(x_r[...], wr[...]), 0.0)
    h_lo[...] = h[:, :_HH]
    h_hi[...] = h[:, _HH:]


def _tc_layer0(xp, sums, c, wl, wr, b):
    return pl.pallas_call(
        _layer0_body,
        grid=(_NBLK,),
        in_specs=[_row_spec(), _half_spec(0), _half_spec(1), _col_spec(),
                  _w_spec(), _w_spec(), _b_spec()],
        out_specs=[_out_half_spec(), _out_half_spec()],
        out_shape=[jax.ShapeDtypeStruct((_NP, _HH), jnp.float32),
                   jax.ShapeDtypeStruct((_NP, _HH), jnp.float32)],
    )(xp, sums, sums, c, wl, wr, b)


def _final_body(h0_lo, h0_hi, h1_lo, h1_hi, sA_lo, sA_hi, cA_r,
                sB_lo, sB_hi, cB_r,
                wl0, wr0, bb0, wl1, wr1, bb1,
                op0, ob0, op1, ob1,
                wq, bq_r, wk, bk_r, wv, bv_r, wo, bo_r,
                w1, b1_r, w2, b2_r, out_r):
    h0 = jnp.concatenate([h0_lo[...], h0_hi[...]], axis=1)
    h1 = jnp.concatenate([h1_lo[...], h1_hi[...]], axis=1)
    aggA = jnp.concatenate([sA_lo[...], sA_hi[...]], axis=1)
    aggA = aggA * (1.0 / jnp.maximum(cA_r[...], 1.0))
    aggB = jnp.concatenate([sB_lo[...], sB_hi[...]], axis=1)
    aggB = aggB * (1.0 / jnp.maximum(cB_r[...], 1.0))
    g0 = jnp.maximum(_dot(aggA, wl0[...]) + bb0[...] + _dot(h0, wr0[...]), 0.0)
    g1 = jnp.maximum(_dot(aggB, wl1[...]) + bb1[...] + _dot(h1, wr1[...]), 0.0)
    xw0 = (_dot(g0, op0[...]) + ob0[...]) * 0.5
    xw1 = (_dot(g1, op1[...]) + ob1[...]) * 0.5
    q0 = _dot(xw0, wq[...]) + bq_r[...]
    q1 = _dot(xw1, wq[...]) + bq_r[...]
    k0 = _dot(xw0, wk[...]) + bk_r[...]
    k1 = _dot(xw1, wk[...]) + bk_r[...]
    v0 = _dot(xw0, wv[...]) + bv_r[...]
    v1 = _dot(xw1, wv[...]) + bv_r[...]
    rr = lax.broadcasted_iota(jnp.int32, (_H, 4), 0)
    cc = lax.broadcasted_iota(jnp.int32, (_H, 4), 1)
    hsel = (rr // 16 == cc).astype(jnp.float32)
    hexp = (lax.broadcasted_iota(jnp.int32, (4, _H), 0) ==
            lax.broadcasted_iota(jnp.int32, (4, _H), 1) // 16
            ).astype(jnp.float32)
    scale = 0.25
    s00 = _dot(q0 * k0, hsel) * scale
    s01 = _dot(q0 * k1, hsel) * scale
    s10 = _dot(q1 * k0, hsel) * scale
    s11 = _dot(q1 * k1, hsel) * scale
    m0 = jnp.maximum(s00, s01)
    e00 = jnp.exp(s00 - m0)
    e01 = jnp.exp(s01 - m0)
    d0 = e00 + e01
    m1 = jnp.maximum(s10, s11)
    e10 = jnp.exp(s10 - m1)
    e11 = jnp.exp(s11 - m1)
    d1 = e10 + e11
    o0 = _dot(e00 / d0, hexp) * v0 + _dot(e01 / d0, hexp) * v1
    o1 = _dot(e10 / d1, hexp) * v0 + _dot(e11 / d1, hexp) * v1
    ao0 = _dot(o0, wo[...]) + bo_r[...]
    ao1 = _dot(o1, wo[...]) + bo_r[...]
    pooled = (ao0 + ao1) * 0.5
    hmid = jnp.maximum(_dot(pooled, w1[...]) + b1_r[...], 0.0)
    out_r[...] = _dot(hmid, w2[...]) + b2_r[...]


def _tc_final(h0_lo, h0_hi, h1_lo, h1_hi, sA, cA, sB, cB,
              wl0, wr0, b0, wl1, wr1, b1,
              op0, ob0, op1, ob1, wq, bq, wk, bk, wv, bv, wo, bo,
              w1, b1m, w2, b2):
    w2spec = pl.BlockSpec((_H, 1), lambda i: (0, 0))
    b2spec = pl.BlockSpec((1, 1), lambda i: (0, 0))
    hs = _out_half_spec
    return pl.pallas_call(
        _final_body,
        grid=(_NBLK,),
        in_specs=[hs(), hs(), hs(), hs(),
                  _half_spec(0), _half_spec(1), _col_spec(),
                  _half_spec(0), _half_spec(1), _col_spec(),
                  _w_spec(), _w_spec(), _b_spec(),
                  _w_spec(), _w_spec(), _b_spec(),
                  _w_spec(), _b_spec(), _w_spec(), _b_spec(),
                  _w_spec(), _b_spec(), _w_spec(), _b_spec(),
                  _w_spec(), _b_spec(), _w_spec(), _b_spec(),
                  _w_spec(), _b_spec(), w2spec, b2spec],
        out_specs=[_col_spec()],
        out_shape=[jax.ShapeDtypeStruct((_NP, 1), jnp.float32)],
    )(h0_lo, h0_hi, h1_lo, h1_hi, sA, sA, cA, sB, sB, cB,
      wl0, wr0, b0, wl1, wr1, b1,
      op0, ob0, op1, ob1, wq, bq, wk, bk, wv, bv, wo, bo,
      w1, b1m, w2, b2)[0]


def _prep_edges(ei):
    src = jnp.concatenate(
        [ei[0], jnp.zeros((_EPAD - _E,), dtype=jnp.int32)]).reshape(_EROWS, 128)
    dst = jnp.concatenate(
        [ei[1], jnp.full((_EPAD - _E,), jnp.int32(_TRASH))]).reshape(
            _EROWS, 128)
    return src, dst


def kernel(x, edge_index_r0, edge_index_r1, conv_Wl, conv_Wr, conv_b,
           outp_W, outp_b, attn_Wq, attn_Wk, attn_Wv, attn_Wo,
           attn_bq, attn_bk, attn_bv, attn_bo,
           mlp_W1, mlp_b1, mlp_W2, mlp_b2):
    seg_c = _make_segsum(True)
    seg_nc = _make_segsum(False)
    s0, d0 = _prep_edges(edge_index_r0)
    s1, d1 = _prep_edges(edge_index_r1)
    xp = jnp.concatenate(
        [x, jnp.zeros((_NP - _N, _H), dtype=jnp.float32)], axis=0)
    x_lo = xp[:, :_HH]
    x_hi = xp[:, _HH:]
    zrow = jnp.zeros((_ZROWS, _HH), dtype=jnp.float32)
    zcnt = jnp.zeros((_TILE_ROWS,), dtype=jnp.float32)
    onesc = jnp.ones((128,), dtype=jnp.float32)

    sums00, cnt0 = seg_c(x_lo, x_hi, s0, d0, zrow, zcnt, onesc)
    sums10, cnt1 = seg_c(x_lo, x_hi, s1, d1, zrow, zcnt, onesc)
    c0 = cnt0.reshape(_NP, 1)
    c1 = cnt1.reshape(_NP, 1)
    h0_lo, h0_hi = _tc_layer0(
        xp, sums00, c0,
        conv_Wl[0, 0].T, conv_Wr[0, 0].T, conv_b[0, 0].reshape(1, _H))
    sums01 = seg_nc(h0_lo, h0_hi, s1, d1, zrow)[0]
    h1_lo, h1_hi = _tc_layer0(
        xp, sums10, c1,
        conv_Wl[1, 0].T, conv_Wr[1, 0].T, conv_b[1, 0].reshape(1, _H))
    sums11 = seg_nc(h1_lo, h1_hi, s0, d0, zrow)[0]

    out = _tc_final(
        h0_lo, h0_hi, h1_lo, h1_hi, sums01, c1, sums11, c0,
        conv_Wl[0, 1].T, conv_Wr[0, 1].T, conv_b[0, 1].reshape(1, _H),
        conv_Wl[1, 1].T, conv_Wr[1, 1].T, conv_b[1, 1].reshape(1, _H),
        outp_W[0].T, outp_b[0].reshape(1, _H),
        outp_W[1].T, outp_b[1].reshape(1, _H),
        attn_Wq.T, attn_bq.reshape(1, _H),
        attn_Wk.T, attn_bk.reshape(1, _H),
        attn_Wv.T, attn_bv.reshape(1, _H),
        attn_Wo.T, attn_bo.reshape(1, _H),
        mlp_W1.T, mlp_b1.reshape(1, _H),
        mlp_W2.T, mlp_b2.reshape(1, 1))
    return out[:_N, 0]


# speedup vs baseline: 9.3893x; 1.7318x over previous
import functools
import jax
import jax.numpy as jnp
from jax import lax
from jax.experimental import pallas as pl
from jax.experimental.pallas import tpu as pltpu
from jax.experimental.pallas import tpu_sc as plsc

_N = 50000
_H = 64
_HH = 32
_E = 800000
_NC = 2
_NS = 16
_NP = 50176
_TRASH = _NP
_ACC_ROWS = _NP + 8
_TILE_ROWS = _NP // _NS
_ZROWS = 64
_EROWS = 6272
_EPAD = _EROWS * 128
_RPT = _EROWS // _NS
_NJ = 2
_OUTER = _RPT // _NJ


def _make_segsum(with_counts):
    mesh = plsc.VectorSubcoreMesh(
        core_axis_name="c", subcore_axis_name="s",
        num_cores=_NC, num_subcores=_NS)
    out_type = [jax.ShapeDtypeStruct((_NC * _NP, _HH), jnp.float32)]
    if with_counts:
        out_type.append(jax.ShapeDtypeStruct((_NP,), jnp.float32))
    scratch = [
        pltpu.VMEM((_NJ, 128), jnp.int32),
        pltpu.VMEM((_NJ, 128), jnp.int32),
        pltpu.VMEM((_NJ, 128), jnp.int32),
        pltpu.VMEM((_NJ, 128), jnp.int32),
        pltpu.VMEM((_NJ, 128, _HH), jnp.float32),
        pltpu.VMEM((_NJ, 128, _HH), jnp.float32),
        pltpu.VMEM((_ZROWS, _HH), jnp.float32),
        pltpu.VMEM_SHARED((_ACC_ROWS, _HH), jnp.float32),
        pltpu.SemaphoreType.DMA,
        pltpu.SemaphoreType.DMA,
        pltpu.SemaphoreType.DMA,
        pltpu.SemaphoreType.DMA,
    ]
    if with_counts:
        scratch += [
            pltpu.VMEM((128,), jnp.float32),
            pltpu.VMEM((_TILE_ROWS,), jnp.float32),
            pltpu.VMEM_SHARED((_ACC_ROWS,), jnp.float32),
        ]

    @functools.partial(pl.kernel, out_type=tuple(out_type), mesh=mesh,
                       scratch_types=scratch,
                       compiler_params=pltpu.CompilerParams(
                           use_tc_tiling_on_sc=False))
    def segsum(*refs):
        if with_counts:
            (h_lo, h_hi, src, dst, zrow, zcnt, onesc, sums, cnt,
             srcbuf0, srcbuf1, dstbuf0, dstbuf1, rows0, rows1, zbuf, acc,
             semi0, semi1, semg0, semg1, ones, zcbuf, cntacc) = refs
        else:
            (h_lo, h_hi, src, dst, zrow, sums,
             srcbuf0, srcbuf1, dstbuf0, dstbuf1, rows0, rows1, zbuf, acc,
             semi0, semi1, semg0, semg1) = refs
        srcbufs = (srcbuf0, srcbuf1)
        dstbufs = (dstbuf0, dstbuf1)
        rowss = (rows0, rows1)
        semis = (semi0, semi1)
        semgs = (semg0, semg1)
        cid = lax.axis_index("c")
        sid = lax.axis_index("s")

        pltpu.sync_copy(zrow, zbuf)
        for k0 in range(_TILE_ROWS // _ZROWS):
            pltpu.sync_copy(
                zbuf, acc.at[pl.ds(sid * _TILE_ROWS + k0 * _ZROWS, _ZROWS)])
        if with_counts:
            pltpu.sync_copy(onesc, ones)
            pltpu.sync_copy(zcnt, zcbuf)
            pltpu.sync_copy(zcbuf,
                            cntacc.at[pl.ds(sid * _TILE_ROWS, _TILE_ROWS)])

        @pl.when(sid == 0)
        def _():
            pltpu.sync_copy(zbuf.at[pl.ds(0, 8)], acc.at[pl.ds(_NP, 8)])
            if with_counts:
                pltpu.sync_copy(zcbuf.at[pl.ds(0, 8)],
                                cntacc.at[pl.ds(_NP, 8)])

        plsc.subcore_barrier()

        def fire_src_idx(g, b):
            row0 = sid * _RPT + g * _NJ
            pltpu.async_copy(src.at[pl.ds(row0, _NJ)], srcbufs[b], semis[b])

        def fire_dst_idx(g, b):
            row0 = sid * _RPT + g * _NJ
            pltpu.async_copy(dst.at[pl.ds(row0, _NJ)], dstbufs[b], semis[b])

        def wait_idx(b):
            pltpu.make_async_copy(
                src.at[pl.ds(0, _NJ)], srcbufs[b], semis[b]).wait()
            pltpu.make_async_copy(
                dst.at[pl.ds(0, _NJ)], dstbufs[b], semis[b]).wait()

        def fire_gather(h, b):
            for r in range(_NJ):
                pltpu.async_copy(h.at[srcbufs[b].at[r]], rowss[b].at[r],
                                 semgs[b])

        def wait_gather(h, b):
            for r in range(_NJ):
                pltpu.make_async_copy(h.at[srcbufs[b].at[r]], rowss[b].at[r],
                                      semgs[b]).wait()

        def run(h):
            fire_src_idx(0, 0)
            fire_dst_idx(0, 0)
            fire_src_idx(1, 1)
            fire_dst_idx(1, 1)
            wait_idx(0)
            fire_gather(h, 0)

            def outer(i, carry):
                for b in range(2):
                    g = i * 2 + b
                    nb = b ^ 1

                    @pl.when(g < _OUTER - 1)
                    def _():
                        wait_idx(nb)
                        fire_gather(h, nb)
                    wait_gather(h, b)

                    @pl.when(g < _OUTER - 2)
                    def _():
                        fire_src_idx(g + 2, b)
                    for r in range(_NJ):
                        pltpu.sync_copy(rowss[b].at[r],
                                        acc.at[dstbufs[b].at[r]], add=True)
                        if with_counts:
                            pltpu.sync_copy(ones, cntacc.at[dstbufs[b].at[r]],
                                            add=True)

                    @pl.when(g < _OUTER - 2)
                    def _():
                        fire_dst_idx(g + 2, b)
                return carry

            lax.fori_loop(0, _OUTER // 2, outer, 0)

        @pl.when(cid == 0)
        def _():
            run(h_lo)

        @pl.when(cid == 1)
        def _():
            run(h_hi)

        plsc.subcore_barrier()
        off = sid * _TILE_ROWS
        pltpu.sync_copy(acc.at[pl.ds(off, _TILE_ROWS)],
                        sums.at[pl.ds(cid * _NP + off, _TILE_ROWS)])
        if with_counts:
            @pl.when(cid == 0)
            def _():
                pltpu.sync_copy(cntacc.at[pl.ds(off, _TILE_ROWS)],
                                cnt.at[pl.ds(off, _TILE_ROWS)])

    return segsum


_BLK = 3584
_NBLK = _NP // _BLK


def _row_spec():
    return pl.BlockSpec((_BLK, _H), lambda i: (i, 0))


def _half_spec(half):
    return pl.BlockSpec((_BLK, _HH), lambda i: (i + half * _NBLK, 0))


def _out_half_spec():
    return pl.BlockSpec((_BLK, _HH), lambda i: (i, 0))


def _col_spec():
    return pl.BlockSpec((_BLK, 1), lambda i: (i, 0))


def _w_spec():
    return pl.BlockSpec((_H, _H), lambda i: (0, 0))


def _b_spec():
    return pl.BlockSpec((1, _H), lambda i: (0, 0))


def _dot(a, b):
    return jnp.dot(a, b, preferred_element_type=jnp.float32)


def _layer0_body(x_r, s_lo, s_hi, c_r, wl, wr, bb, h_lo, h_hi):
    agg = jnp.concatenate([s_lo[...], s_hi[...]], axis=1)
    agg = agg * (1.0 / jnp.maximum(c_r[...], 1.0))
    h = jnp.maximum(
        _dot(agg, wl[...]) + bb[...] + _dot(x_r[...], wr[...]), 0.0)
    h_lo[...] = h[:, :_HH]
    h_hi[...] = h[:, _HH:]


def _tc_layer0(xp, sums, c, wl, wr, b):
    return pl.pallas_call(
        _layer0_body,
        grid=(_NBLK,),
        in_specs=[_row_spec(), _half_spec(0), _half_spec(1), _col_spec(),
                  _w_spec(), _w_spec(), _b_spec()],
        out_specs=[_out_half_spec(), _out_half_spec()],
        out_shape=[jax.ShapeDtypeStruct((_NP, _HH), jnp.float32),
                   jax.ShapeDtypeStruct((_NP, _HH), jnp.float32)],
    )(xp, sums, sums, c, wl, wr, b)


def _final_body(h0_lo, h0_hi, h1_lo, h1_hi, sA_lo, sA_hi, cA_r,
                sB_lo, sB_hi, cB_r,
                wl0, wr0, bb0, wl1, wr1, bb1,
                op0, ob0, op1, ob1,
                wq, bq_r, wk, bk_r, wv, bv_r, wo, bo_r,
                w1, b1_r, w2, b2_r, out_r):
    h0 = jnp.concatenate([h0_lo[...], h0_hi[...]], axis=1)
    h1 = jnp.concatenate([h1_lo[...], h1_hi[...]], axis=1)
    aggA = jnp.concatenate([sA_lo[...], sA_hi[...]], axis=1)
    aggA = aggA * (1.0 / jnp.maximum(cA_r[...], 1.0))
    aggB = jnp.concatenate([sB_lo[...], sB_hi[...]], axis=1)
    aggB = aggB * (1.0 / jnp.maximum(cB_r[...], 1.0))
    g0 = jnp.maximum(_dot(aggA, wl0[...]) + bb0[...] + _dot(h0, wr0[...]), 0.0)
    g1 = jnp.maximum(_dot(aggB, wl1[...]) + bb1[...] + _dot(h1, wr1[...]), 0.0)
    xw0 = (_dot(g0, op0[...]) + ob0[...]) * 0.5
    xw1 = (_dot(g1, op1[...]) + ob1[...]) * 0.5
    q0 = _dot(xw0, wq[...]) + bq_r[...]
    q1 = _dot(xw1, wq[...]) + bq_r[...]
    k0 = _dot(xw0, wk[...]) + bk_r[...]
    k1 = _dot(xw1, wk[...]) + bk_r[...]
    v0 = _dot(xw0, wv[...]) + bv_r[...]
    v1 = _dot(xw1, wv[...]) + bv_r[...]
    rr = lax.broadcasted_iota(jnp.int32, (_H, 4), 0)
    cc = lax.broadcasted_iota(jnp.int32, (_H, 4), 1)
    hsel = (rr // 16 == cc).astype(jnp.float32)
    hexp = (lax.broadcasted_iota(jnp.int32, (4, _H), 0) ==
            lax.broadcasted_iota(jnp.int32, (4, _H), 1) // 16
            ).astype(jnp.float32)
    scale = 0.25
    s00 = _dot(q0 * k0, hsel) * scale
    s01 = _dot(q0 * k1, hsel) * scale
    s10 = _dot(q1 * k0, hsel) * scale
    s11 = _dot(q1 * k1, hsel) * scale
    m0 = jnp.maximum(s00, s01)
    e00 = jnp.exp(s00 - m0)
    e01 = jnp.exp(s01 - m0)
    d0 = e00 + e01
    m1 = jnp.maximum(s10, s11)
    e10 = jnp.exp(s10 - m1)
    e11 = jnp.exp(s11 - m1)
    d1 = e10 + e11
    o0 = _dot(e00 / d0, hexp) * v0 + _dot(e01 / d0, hexp) * v1
    o1 = _dot(e10 / d1, hexp) * v0 + _dot(e11 / d1, hexp) * v1
    ao0 = _dot(o0, wo[...]) + bo_r[...]
    ao1 = _dot(o1, wo[...]) + bo_r[...]
    pooled = (ao0 + ao1) * 0.5
    hmid = jnp.maximum(_dot(pooled, w1[...]) + b1_r[...], 0.0)
    out_r[...] = _dot(hmid, w2[...]) + b2_r[...]


def _tc_final(h0_lo, h0_hi, h1_lo, h1_hi, sA, cA, sB, cB,
              wl0, wr0, b0, wl1, wr1, b1,
              op0, ob0, op1, ob1, wq, bq, wk, bk, wv, bv, wo, bo,
              w1, b1m, w2, b2):
    w2spec = pl.BlockSpec((_H, 1), lambda i: (0, 0))
    b2spec = pl.BlockSpec((1, 1), lambda i: (0, 0))
    hs = _out_half_spec
    return pl.pallas_call(
        _final_body,
        grid=(_NBLK,),
        in_specs=[hs(), hs(), hs(), hs(),
                  _half_spec(0), _half_spec(1), _col_spec(),
                  _half_spec(0), _half_spec(1), _col_spec(),
                  _w_spec(), _w_spec(), _b_spec(),
                  _w_spec(), _w_spec(), _b_spec(),
                  _w_spec(), _b_spec(), _w_spec(), _b_spec(),
                  _w_spec(), _b_spec(), _w_spec(), _b_spec(),
                  _w_spec(), _b_spec(), _w_spec(), _b_spec(),
                  _w_spec(), _b_spec(), w2spec, b2spec],
        out_specs=[_col_spec()],
        out_shape=[jax.ShapeDtypeStruct((_NP, 1), jnp.float32)],
    )(h0_lo, h0_hi, h1_lo, h1_hi, sA, sA, cA, sB, sB, cB,
      wl0, wr0, b0, wl1, wr1, b1,
      op0, ob0, op1, ob1, wq, bq, wk, bk, wv, bv, wo, bo,
      w1, b1m, w2, b2)[0]


def _prep_edges(ei):
    src = jnp.concatenate(
        [ei[0], jnp.zeros((_EPAD - _E,), dtype=jnp.int32)]).reshape(_EROWS, 128)
    dst = jnp.concatenate(
        [ei[1], jnp.full((_EPAD - _E,), jnp.int32(_TRASH))]).reshape(
            _EROWS, 128)
    return src, dst


def kernel(x, edge_index_r0, edge_index_r1, conv_Wl, conv_Wr, conv_b,
           outp_W, outp_b, attn_Wq, attn_Wk, attn_Wv, attn_Wo,
           attn_bq, attn_bk, attn_bv, attn_bo,
           mlp_W1, mlp_b1, mlp_W2, mlp_b2):
    seg_c = _make_segsum(True)
    seg_nc = _make_segsum(False)
    s0, d0 = _prep_edges(edge_index_r0)
    s1, d1 = _prep_edges(edge_index_r1)
    xp = jnp.concatenate(
        [x, jnp.zeros((_NP - _N, _H), dtype=jnp.float32)], axis=0)
    x_lo = xp[:, :_HH]
    x_hi = xp[:, _HH:]
    zrow = jnp.zeros((_ZROWS, _HH), dtype=jnp.float32)
    zcnt = jnp.zeros((_TILE_ROWS,), dtype=jnp.float32)
    onesc = jnp.ones((128,), dtype=jnp.float32)

    sums00, cnt0 = seg_c(x_lo, x_hi, s0, d0, zrow, zcnt, onesc)
    sums10, cnt1 = seg_c(x_lo, x_hi, s1, d1, zrow, zcnt, onesc)
    c0 = cnt0.reshape(_NP, 1)
    c1 = cnt1.reshape(_NP, 1)
    h0_lo, h0_hi = _tc_layer0(
        xp, sums00, c0,
        conv_Wl[0, 0].T, conv_Wr[0, 0].T, conv_b[0, 0].reshape(1, _H))
    sums01 = seg_nc(h0_lo, h0_hi, s1, d1, zrow)[0]
    h1_lo, h1_hi = _tc_layer0(
        xp, sums10, c1,
        conv_Wl[1, 0].T, conv_Wr[1, 0].T, conv_b[1, 0].reshape(1, _H))
    sums11 = seg_nc(h1_lo, h1_hi, s0, d0, zrow)[0]

    out = _tc_final(
        h0_lo, h0_hi, h1_lo, h1_hi, sums01, c1, sums11, c0,
        conv_Wl[0, 1].T, conv_Wr[0, 1].T, conv_b[0, 1].reshape(1, _H),
        conv_Wl[1, 1].T, conv_Wr[1, 1].T, conv_b[1, 1].reshape(1, _H),
        outp_W[0].T, outp_b[0].reshape(1, _H),
        outp_W[1].T, outp_b[1].reshape(1, _H),
        attn_Wq.T, attn_bq.reshape(1, _H),
        attn_Wk.T, attn_bk.reshape(1, _H),
        attn_Wv.T, attn_bv.reshape(1, _H),
        attn_Wo.T, attn_bo.reshape(1, _H),
        mlp_W1.T, mlp_b1.reshape(1, _H),
        mlp_W2.T, mlp_b2.reshape(1, 1))
    return out[:_N, 0]


# speedup vs baseline: 9.7281x; 1.0361x over previous
import functools
import jax
import jax.numpy as jnp
from jax import lax
from jax.experimental import pallas as pl
from jax.experimental.pallas import tpu as pltpu
from jax.experimental.pallas import tpu_sc as plsc

_N = 50000
_H = 64
_HH = 32
_E = 800000
_NC = 2
_NS = 16
_NP = 50176
_TRASH = _NP
_ACC_ROWS = _NP + 8
_TILE_ROWS = _NP // _NS
_ZROWS = 64
_EROWS = 6272
_EPAD = _EROWS * 128
_RPT = _EROWS // _NS
_NJ = 2
_OUTER = _RPT // _NJ


def _make_segsum(with_counts):
    mesh = plsc.VectorSubcoreMesh(
        core_axis_name="c", subcore_axis_name="s",
        num_cores=_NC, num_subcores=_NS)
    out_type = [jax.ShapeDtypeStruct((_NC * _NP, _HH), jnp.float32)]
    if with_counts:
        out_type.append(jax.ShapeDtypeStruct((_NP,), jnp.float32))
    scratch = [
        pltpu.VMEM((_NJ * 128,), jnp.int32),
        pltpu.VMEM((_NJ * 128,), jnp.int32),
        pltpu.VMEM((_NJ * 128,), jnp.int32),
        pltpu.VMEM((_NJ * 128,), jnp.int32),
        pltpu.VMEM((_NJ * 128, _HH), jnp.float32),
        pltpu.VMEM((_NJ * 128, _HH), jnp.float32),
        pltpu.VMEM((_ZROWS, _HH), jnp.float32),
        pltpu.VMEM_SHARED((_ACC_ROWS, _HH), jnp.float32),
        pltpu.SemaphoreType.DMA,
        pltpu.SemaphoreType.DMA,
        pltpu.SemaphoreType.DMA,
        pltpu.SemaphoreType.DMA,
    ]
    if with_counts:
        scratch += [
            pltpu.VMEM((_NJ * 128,), jnp.float32),
            pltpu.VMEM((_TILE_ROWS,), jnp.float32),
            pltpu.VMEM_SHARED((_ACC_ROWS,), jnp.float32),
        ]

    @functools.partial(pl.kernel, out_type=tuple(out_type), mesh=mesh,
                       scratch_types=scratch,
                       compiler_params=pltpu.CompilerParams(
                           use_tc_tiling_on_sc=False))
    def segsum(*refs):
        if with_counts:
            (h_lo, h_hi, src, dst, zrow, zcnt, onesc, sums, cnt,
             srcbuf0, srcbuf1, dstbuf0, dstbuf1, rows0, rows1, zbuf, acc,
             semi0, semi1, semg0, semg1, ones, zcbuf, cntacc) = refs
        else:
            (h_lo, h_hi, src, dst, zrow, sums,
             srcbuf0, srcbuf1, dstbuf0, dstbuf1, rows0, rows1, zbuf, acc,
             semi0, semi1, semg0, semg1) = refs
        srcbufs = (srcbuf0, srcbuf1)
        dstbufs = (dstbuf0, dstbuf1)
        rowss = (rows0, rows1)
        semis = (semi0, semi1)
        semgs = (semg0, semg1)
        cid = lax.axis_index("c")
        sid = lax.axis_index("s")

        pltpu.sync_copy(zrow, zbuf)
        for k0 in range(_TILE_ROWS // _ZROWS):
            pltpu.sync_copy(
                zbuf, acc.at[pl.ds(sid * _TILE_ROWS + k0 * _ZROWS, _ZROWS)])
        if with_counts:
            pltpu.sync_copy(onesc, ones)
            pltpu.sync_copy(zcnt, zcbuf)
            pltpu.sync_copy(zcbuf,
                            cntacc.at[pl.ds(sid * _TILE_ROWS, _TILE_ROWS)])

        @pl.when(sid == 0)
        def _():
            pltpu.sync_copy(zbuf.at[pl.ds(0, 8)], acc.at[pl.ds(_NP, 8)])
            if with_counts:
                pltpu.sync_copy(zcbuf.at[pl.ds(0, 8)],
                                cntacc.at[pl.ds(_NP, 8)])

        plsc.subcore_barrier()

        def fire_src_idx(g, b):
            e0 = (sid * _RPT + g * _NJ) * 128
            pltpu.async_copy(src.at[pl.ds(e0, _NJ * 128)], srcbufs[b],
                             semis[b])

        def fire_dst_idx(g, b):
            e0 = (sid * _RPT + g * _NJ) * 128
            pltpu.async_copy(dst.at[pl.ds(e0, _NJ * 128)], dstbufs[b],
                             semis[b])

        def wait_idx(b):
            pltpu.make_async_copy(
                src.at[pl.ds(0, _NJ * 128)], srcbufs[b], semis[b]).wait()
            pltpu.make_async_copy(
                dst.at[pl.ds(0, _NJ * 128)], dstbufs[b], semis[b]).wait()

        def fire_gather(h, b):
            pltpu.async_copy(h.at[srcbufs[b]], rowss[b], semgs[b])

        def wait_gather(h, b):
            pltpu.make_async_copy(h.at[srcbufs[b]], rowss[b],
                                  semgs[b]).wait()

        def run(h):
            fire_src_idx(0, 0)
            fire_dst_idx(0, 0)
            fire_src_idx(1, 1)
            fire_dst_idx(1, 1)
            wait_idx(0)
            fire_gather(h, 0)

            def outer(i, carry):
                for b in range(2):
                    g = i * 2 + b
                    nb = b ^ 1

                    @pl.when(g < _OUTER - 1)
                    def _():
                        wait_idx(nb)
                        fire_gather(h, nb)
                    wait_gather(h, b)

                    @pl.when(g < _OUTER - 2)
                    def _():
                        fire_src_idx(g + 2, b)
                    pltpu.sync_copy(rowss[b], acc.at[dstbufs[b]], add=True)
                    if with_counts:
                        pltpu.sync_copy(ones, cntacc.at[dstbufs[b]],
                                        add=True)

                    @pl.when(g < _OUTER - 2)
                    def _():
                        fire_dst_idx(g + 2, b)
                return carry

            lax.fori_loop(0, _OUTER // 2, outer, 0)

        @pl.when(cid == 0)
        def _():
            run(h_lo)

        @pl.when(cid == 1)
        def _():
            run(h_hi)

        plsc.subcore_barrier()
        off = sid * _TILE_ROWS
        pltpu.sync_copy(acc.at[pl.ds(off, _TILE_ROWS)],
                        sums.at[pl.ds(cid * _NP + off, _TILE_ROWS)])
        if with_counts:
            @pl.when(cid == 0)
            def _():
                pltpu.sync_copy(cntacc.at[pl.ds(off, _TILE_ROWS)],
                                cnt.at[pl.ds(off, _TILE_ROWS)])

    return segsum


_BLK = 3584
_NBLK = _NP // _BLK


def _row_spec():
    return pl.BlockSpec((_BLK, _H), lambda i: (i, 0))


def _half_spec(half):
    return pl.BlockSpec((_BLK, _HH), lambda i: (i + half * _NBLK, 0))


def _out_half_spec():
    return pl.BlockSpec((_BLK, _HH), lambda i: (i, 0))


def _col_spec():
    return pl.BlockSpec((_BLK, 1), lambda i: (i, 0))


def _w_spec():
    return pl.BlockSpec((_H, _H), lambda i: (0, 0))


def _b_spec():
    return pl.BlockSpec((1, _H), lambda i: (0, 0))


def _dot(a, b):
    return jnp.dot(a, b, preferred_element_type=jnp.float32)


def _layer0_body(x_r, s_lo, s_hi, c_r, wl, wr, bb, h_lo, h_hi):
    agg = jnp.concatenate([s_lo[...], s_hi[...]], axis=1)
    agg = agg * (1.0 / jnp.maximum(c_r[...], 1.0))
    h = jnp.maximum(
        _dot(agg, wl[...]) + bb[...] + _dot(x_r[...], wr[...]), 0.0)
    h_lo[...] = h[:, :_HH]
    h_hi[...] = h[:, _HH:]


def _tc_layer0(xp, sums, c, wl, wr, b):
    return pl.pallas_call(
        _layer0_body,
        grid=(_NBLK,),
        in_specs=[_row_spec(), _half_spec(0), _half_spec(1), _col_spec(),
                  _w_spec(), _w_spec(), _b_spec()],
        out_specs=[_out_half_spec(), _out_half_spec()],
        out_shape=[jax.ShapeDtypeStruct((_NP, _HH), jnp.float32),
                   jax.ShapeDtypeStruct((_NP, _HH), jnp.float32)],
    )(xp, sums, sums, c, wl, wr, b)


def _final_body(h0_lo, h0_hi, h1_lo, h1_hi, sA_lo, sA_hi, cA_r,
                sB_lo, sB_hi, cB_r,
                wl0, wr0, bb0, wl1, wr1, bb1,
                op0, ob0, op1, ob1,
                wq, bq_r, wk, bk_r, wv, bv_r, wo, bo_r,
                w1, b1_r, w2, b2_r, out_r):
    h0 = jnp.concatenate([h0_lo[...], h0_hi[...]], axis=1)
    h1 = jnp.concatenate([h1_lo[...], h1_hi[...]], axis=1)
    aggA = jnp.concatenate([sA_lo[...], sA_hi[...]], axis=1)
    aggA = aggA * (1.0 / jnp.maximum(cA_r[...], 1.0))
    aggB = jnp.concatenate([sB_lo[...], sB_hi[...]], axis=1)
    aggB = aggB * (1.0 / jnp.maximum(cB_r[...], 1.0))
    g0 = jnp.maximum(_dot(aggA, wl0[...]) + bb0[...] + _dot(h0, wr0[...]), 0.0)
    g1 = jnp.maximum(_dot(aggB, wl1[...]) + bb1[...] + _dot(h1, wr1[...]), 0.0)
    xw0 = (_dot(g0, op0[...]) + ob0[...]) * 0.5
    xw1 = (_dot(g1, op1[...]) + ob1[...]) * 0.5
    q0 = _dot(xw0, wq[...]) + bq_r[...]
    q1 = _dot(xw1, wq[...]) + bq_r[...]
    k0 = _dot(xw0, wk[...]) + bk_r[...]
    k1 = _dot(xw1, wk[...]) + bk_r[...]
    v0 = _dot(xw0, wv[...]) + bv_r[...]
    v1 = _dot(xw1, wv[...]) + bv_r[...]
    rr = lax.broadcasted_iota(jnp.int32, (_H, 4), 0)
    cc = lax.broadcasted_iota(jnp.int32, (_H, 4), 1)
    hsel = (rr // 16 == cc).astype(jnp.float32)
    hexp = (lax.broadcasted_iota(jnp.int32, (4, _H), 0) ==
            lax.broadcasted_iota(jnp.int32, (4, _H), 1) // 16
            ).astype(jnp.float32)
    scale = 0.25
    s00 = _dot(q0 * k0, hsel) * scale
    s01 = _dot(q0 * k1, hsel) * scale
    s10 = _dot(q1 * k0, hsel) * scale
    s11 = _dot(q1 * k1, hsel) * scale
    m0 = jnp.maximum(s00, s01)
    e00 = jnp.exp(s00 - m0)
    e01 = jnp.exp(s01 - m0)
    d0 = e00 + e01
    m1 = jnp.maximum(s10, s11)
    e10 = jnp.exp(s10 - m1)
    e11 = jnp.exp(s11 - m1)
    d1 = e10 + e11
    o0 = _dot(e00 / d0, hexp) * v0 + _dot(e01 / d0, hexp) * v1
    o1 = _dot(e10 / d1, hexp) * v0 + _dot(e11 / d1, hexp) * v1
    ao0 = _dot(o0, wo[...]) + bo_r[...]
    ao1 = _dot(o1, wo[...]) + bo_r[...]
    pooled = (ao0 + ao1) * 0.5
    hmid = jnp.maximum(_dot(pooled, w1[...]) + b1_r[...], 0.0)
    out_r[...] = _dot(hmid, w2[...]) + b2_r[...]


def _tc_final(h0_lo, h0_hi, h1_lo, h1_hi, sA, cA, sB, cB,
              wl0, wr0, b0, wl1, wr1, b1,
              op0, ob0, op1, ob1, wq, bq, wk, bk, wv, bv, wo, bo,
              w1, b1m, w2, b2):
    w2spec = pl.BlockSpec((_H, 1), lambda i: (0, 0))
    b2spec = pl.BlockSpec((1, 1), lambda i: (0, 0))
    hs = _out_half_spec
    return pl.pallas_call(
        _final_body,
        grid=(_NBLK,),
        in_specs=[hs(), hs(), hs(), hs(),
                  _half_spec(0), _half_spec(1), _col_spec(),
                  _half_spec(0), _half_spec(1), _col_spec(),
                  _w_spec(), _w_spec(), _b_spec(),
                  _w_spec(), _w_spec(), _b_spec(),
                  _w_spec(), _b_spec(), _w_spec(), _b_spec(),
                  _w_spec(), _b_spec(), _w_spec(), _b_spec(),
                  _w_spec(), _b_spec(), _w_spec(), _b_spec(),
                  _w_spec(), _b_spec(), w2spec, b2spec],
        out_specs=[_col_spec()],
        out_shape=[jax.ShapeDtypeStruct((_NP, 1), jnp.float32)],
    )(h0_lo, h0_hi, h1_lo, h1_hi, sA, sA, cA, sB, sB, cB,
      wl0, wr0, b0, wl1, wr1, b1,
      op0, ob0, op1, ob1, wq, bq, wk, bk, wv, bv, wo, bo,
      w1, b1m, w2, b2)[0]


def _prep_edges(ei):
    src = jnp.concatenate(
        [ei[0], jnp.zeros((_EPAD - _E,), dtype=jnp.int32)])
    dst = jnp.concatenate(
        [ei[1], jnp.full((_EPAD - _E,), jnp.int32(_TRASH))])
    return src, dst


def kernel(x, edge_index_r0, edge_index_r1, conv_Wl, conv_Wr, conv_b,
           outp_W, outp_b, attn_Wq, attn_Wk, attn_Wv, attn_Wo,
           attn_bq, attn_bk, attn_bv, attn_bo,
           mlp_W1, mlp_b1, mlp_W2, mlp_b2):
    seg_c = _make_segsum(True)
    seg_nc = _make_segsum(False)
    s0, d0 = _prep_edges(edge_index_r0)
    s1, d1 = _prep_edges(edge_index_r1)
    xp = jnp.concatenate(
        [x, jnp.zeros((_NP - _N, _H), dtype=jnp.float32)], axis=0)
    x_lo = xp[:, :_HH]
    x_hi = xp[:, _HH:]
    zrow = jnp.zeros((_ZROWS, _HH), dtype=jnp.float32)
    zcnt = jnp.zeros((_TILE_ROWS,), dtype=jnp.float32)
    onesc = jnp.ones((_NJ * 128,), dtype=jnp.float32)

    sums00, cnt0 = seg_c(x_lo, x_hi, s0, d0, zrow, zcnt, onesc)
    sums10, cnt1 = seg_c(x_lo, x_hi, s1, d1, zrow, zcnt, onesc)
    c0 = cnt0.reshape(_NP, 1)
    c1 = cnt1.reshape(_NP, 1)
    h0_lo, h0_hi = _tc_layer0(
        xp, sums00, c0,
        conv_Wl[0, 0].T, conv_Wr[0, 0].T, conv_b[0, 0].reshape(1, _H))
    sums01 = seg_nc(h0_lo, h0_hi, s1, d1, zrow)[0]
    h1_lo, h1_hi = _tc_layer0(
        xp, sums10, c1,
        conv_Wl[1, 0].T, conv_Wr[1, 0].T, conv_b[1, 0].reshape(1, _H))
    sums11 = seg_nc(h1_lo, h1_hi, s0, d0, zrow)[0]

    out = _tc_final(
        h0_lo, h0_hi, h1_lo, h1_hi, sums01, c1, sums11, c0,
        conv_Wl[0, 1].T, conv_Wr[0, 1].T, conv_b[0, 1].reshape(1, _H),
        conv_Wl[1, 1].T, conv_Wr[1, 1].T, conv_b[1, 1].reshape(1, _H),
        outp_W[0].T, outp_b[0].reshape(1, _H),
        outp_W[1].T, outp_b[1].reshape(1, _H),
        attn_Wq.T, attn_bq.reshape(1, _H),
        attn_Wk.T, attn_bk.reshape(1, _H),
        attn_Wv.T, attn_bv.reshape(1, _H),
        attn_Wo.T, attn_bo.reshape(1, _H),
        mlp_W1.T, mlp_b1.reshape(1, _H),
        mlp_W2.T, mlp_b2.reshape(1, 1))
    return out[:_N, 0]
